# baseline (speedup 1.0000x reference)
import jax
import jax.numpy as jnp
from jax import lax
from jax.experimental import pallas as pl
from jax.experimental.pallas import tpu as pltpu

_B = 16384
_P = 12
_V = 28
_D = 24
_BB = 1024
_NBLK = _B // _BB


def _tc_body(x_ref, word_ref, pos_ref, out_hbm, scratch, sem):
    i = pl.program_id(0)
    x = x_ref[...]
    w = word_ref[...]
    iota_v = lax.broadcasted_iota(jnp.int32, (1, _V), 1)
    for p in range(_P):
        cp = w + pos_ref[pl.ds(p, 1), :]
        xp = x[:, p:p + 1]
        oh = (xp == iota_v).astype(jnp.float32)
        scratch[:, p, :] = lax.dot_general(
            oh, cp, dimension_numbers=(((1,), (0,)), ((), ())),
            preferred_element_type=jnp.float32)
    cp_out = pltpu.make_async_copy(
        scratch, out_hbm.at[pl.ds(i * _BB, _BB)], sem)
    cp_out.start()
    cp_out.wait()


def kernel(X, word_table, pos_table):
    grid = (_NBLK,)
    return pl.pallas_call(
        _tc_body,
        grid=grid,
        in_specs=[
            pl.BlockSpec((_BB, _P), lambda i: (i, 0)),
            pl.BlockSpec((_V, _D), lambda i: (0, 0)),
            pl.BlockSpec((_P, _D), lambda i: (0, 0)),
        ],
        out_specs=pl.BlockSpec(memory_space=pl.ANY),
        out_shape=jax.ShapeDtypeStruct((_B, _P, _D), jnp.float32),
        scratch_shapes=[
            pltpu.VMEM((_BB, _P, _D), jnp.float32),
            pltpu.SemaphoreType.DMA,
        ],
    )(X.astype(jnp.int32), word_table, pos_table)


# speedup vs baseline: 1.2513x; 1.2513x over previous
import jax
import jax.numpy as jnp
from jax import lax
from jax.experimental import pallas as pl
from jax.experimental.pallas import tpu as pltpu

_B = 16384
_P = 12
_V = 28
_D = 24
_BB = 1024
_NBLK = _B // _BB


def _copy(scratch, out_hbm, sem, slot, blk):
    return pltpu.make_async_copy(
        scratch.at[slot], out_hbm.at[pl.ds(blk * _BB, _BB)], sem.at[slot])


def _tc_body(x_ref, word_ref, pos_ref, out_hbm, scratch, sem):
    i = pl.program_id(0)
    slot = lax.rem(i, 2)

    @pl.when(i >= 2)
    def _wait_prev():
        _copy(scratch, out_hbm, sem, slot, i - 2).wait()

    x = x_ref[...]
    w = word_ref[...]
    iota_v = lax.broadcasted_iota(jnp.int32, (1, _V), 1)
    for p in range(_P):
        cp = w + pos_ref[pl.ds(p, 1), :]
        xp = x[:, p:p + 1]
        oh = (xp == iota_v).astype(jnp.float32)
        scratch[slot, :, p, :] = lax.dot_general(
            oh, cp, dimension_numbers=(((1,), (0,)), ((), ())),
            preferred_element_type=jnp.float32)
    _copy(scratch, out_hbm, sem, slot, i).start()

    @pl.when(i == _NBLK - 1)
    def _drain():
        _copy(scratch, out_hbm, sem, 1 - slot, i - 1).wait()
        _copy(scratch, out_hbm, sem, slot, i).wait()


def kernel(X, word_table, pos_table):
    grid = (_NBLK,)
    return pl.pallas_call(
        _tc_body,
        grid=grid,
        in_specs=[
            pl.BlockSpec((_BB, _P), lambda i: (i, 0)),
            pl.BlockSpec((_V, _D), lambda i: (0, 0)),
            pl.BlockSpec((_P, _D), lambda i: (0, 0)),
        ],
        out_specs=pl.BlockSpec(memory_space=pl.ANY),
        out_shape=jax.ShapeDtypeStruct((_B, _P, _D), jnp.float32),
        scratch_shapes=[
            pltpu.VMEM((2, _BB, _P, _D), jnp.float32),
            pltpu.SemaphoreType.DMA((2,)),
        ],
    )(X.astype(jnp.int32), word_table, pos_table)


# speedup vs baseline: 1.4911x; 1.1916x over previous
import jax
import jax.numpy as jnp
from jax import lax
from jax.experimental import pallas as pl
from jax.experimental.pallas import tpu as pltpu

_B = 16384
_P = 12
_V = 28
_D = 24
_BB = 1024
_NBLK = _B // _BB


def _copy(scratch, out_hbm, sem, slot, blk):
    return pltpu.make_async_copy(
        scratch.at[slot], out_hbm.at[pl.ds(blk * _BB, _BB)], sem.at[slot])


def _tc_body(x_ref, word_ref, pos_ref, out_hbm, scratch, sem):
    i = pl.program_id(0)
    slot = lax.rem(i, 2)

    @pl.when(i >= 2)
    def _wait_prev():
        _copy(scratch, out_hbm, sem, slot, i - 2).wait()

    x = x_ref[...]
    w = word_ref[...]
    iota_v = lax.broadcasted_iota(jnp.int32, (1, _V), 1)
    res = []
    for p in range(_P):
        cp = w + pos_ref[pl.ds(p, 1), :]
        xp = x[:, p:p + 1]
        oh = (xp == iota_v).astype(jnp.float32)
        res.append(lax.dot_general(
            oh, cp, dimension_numbers=(((1,), (0,)), ((), ())),
            preferred_element_type=jnp.float32))
    res_all = jnp.stack(res, axis=0)
    for g in range(_BB // 8):
        blk = res_all[:, g * 8:(g + 1) * 8, :]
        scratch[slot, pl.ds(g * 8, 8), :, :] = jnp.swapaxes(blk, 0, 1)
    _copy(scratch, out_hbm, sem, slot, i).start()

    @pl.when(i == _NBLK - 1)
    def _drain():
        _copy(scratch, out_hbm, sem, 1 - slot, i - 1).wait()
        _copy(scratch, out_hbm, sem, slot, i).wait()


def kernel(X, word_table, pos_table):
    grid = (_NBLK,)
    return pl.pallas_call(
        _tc_body,
        grid=grid,
        in_specs=[
            pl.BlockSpec((_BB, _P), lambda i: (i, 0)),
            pl.BlockSpec((_V, _D), lambda i: (0, 0)),
            pl.BlockSpec((_P, _D), lambda i: (0, 0)),
        ],
        out_specs=pl.BlockSpec(memory_space=pl.ANY),
        out_shape=jax.ShapeDtypeStruct((_B, _P, _D), jnp.float32),
        scratch_shapes=[
            pltpu.VMEM((2, _BB, _P, _D), jnp.float32),
            pltpu.SemaphoreType.DMA((2,)),
        ],
    )(X.astype(jnp.int32), word_table, pos_table)
